# baseline (speedup 1.0000x reference)
import functools
import math

import jax
import jax.numpy as jnp
from jax import lax
from jax.experimental import pallas as pl
from jax.experimental.pallas import tpu as pltpu
from jax.experimental.pallas import tpu_sc as plsc

D_MODEL = 128
SCALE = math.sqrt(float(D_MODEL))


def kernel(x, table):
    B0, B1 = x.shape
    V, D = table.shape
    info = plsc.get_sparse_core_info()
    NC, NS, L = info.num_cores, info.num_subcores, info.num_lanes
    NW = NC * NS
    total = B0 * B1
    CH = 128
    NJ = total // (NW * CH)
    assert NJ * CH * NW == total and D % L == 0 and NJ >= 4 and NJ % 2 == 0

    xr = x.reshape(NW, NJ, CH).astype(jnp.int32)
    mesh = plsc.VectorSubcoreMesh(core_axis_name="c", subcore_axis_name="s")

    @functools.partial(
        pl.kernel,
        mesh=mesh,
        out_type=jax.ShapeDtypeStruct((NW, NJ, CH, D), jnp.float32),
        scratch_types=[
            pltpu.VMEM((NJ, CH), jnp.int32),
            pltpu.VMEM((2, CH, D), jnp.float32),
            pltpu.VMEM((2, CH, D), jnp.float32),
            pltpu.SemaphoreType.DMA,
            pltpu.SemaphoreType.DMA,
            pltpu.SemaphoreType.DMA,
            pltpu.SemaphoreType.DMA,
        ],
    )
    def emb_kernel(x_hbm, table_hbm, out_hbm, idx_v, rows_in, rows_out,
                   sem_g0, sem_g1, sem_s0, sem_s1):
        c = lax.axis_index("c")
        s = lax.axis_index("s")
        wid = s * NC + c
        pltpu.sync_copy(x_hbm.at[wid], idx_v)
        sem_g = (sem_g0, sem_g1)
        sem_s = (sem_s0, sem_s1)

        def g_copy(jj, b):
            return pltpu.make_async_copy(
                table_hbm.at[idx_v.at[jj]], rows_in.at[b], sem_g[b])

        def s_copy(jj, b):
            return pltpu.make_async_copy(
                rows_out.at[b], out_hbm.at[wid, jj], sem_s[b])

        def scale(b):
            @plsc.parallel_loop(0, CH, unroll=4)
            def _(r):
                for t in range(D // 16):
                    sl = pl.ds(t * 16, 16)
                    rows_out[b, r, sl] = rows_in[b, r, sl] * SCALE

        g_copy(0, 0).start()
        g_copy(0, 0).wait()
        g_copy(1, 1).start()
        scale(0)
        s_copy(0, 0).start()
        g_copy(1, 1).wait()
        g_copy(2, 0).start()
        scale(1)
        s_copy(1, 1).start()

        def pair(p, carry):
            jj0 = 2 * p + 2
            for b in range(2):
                jj = jj0 + b
                g_copy(jj, b).wait()
                s_copy(jj - 2, b).wait()
                g_copy(jj + 1, 1 - b).start()
                scale(b)
                s_copy(jj, b).start()
            return carry
        lax.fori_loop(0, (NJ - 4) // 2, pair, 0)

        g_copy(NJ - 2, 0).wait()
        s_copy(NJ - 4, 0).wait()
        g_copy(NJ - 1, 1).start()
        scale(0)
        s_copy(NJ - 2, 0).start()
        g_copy(NJ - 1, 1).wait()
        s_copy(NJ - 3, 1).wait()
        scale(1)
        s_copy(NJ - 1, 1).start()
        s_copy(NJ - 2, 0).wait()
        s_copy(NJ - 1, 1).wait()

    out = emb_kernel(xr, table)
    return out.reshape(B0, B1, D)


# speedup vs baseline: 1.3051x; 1.3051x over previous
import functools
import math

import jax
import jax.numpy as jnp
from jax import lax
from jax.experimental import pallas as pl
from jax.experimental.pallas import tpu as pltpu
from jax.experimental.pallas import tpu_sc as plsc

D_MODEL = 128
SCALE = math.sqrt(float(D_MODEL))


def kernel(x, table):
    B0, B1 = x.shape
    V, D = table.shape
    info = plsc.get_sparse_core_info()
    NC, NS, L = info.num_cores, info.num_subcores, info.num_lanes
    NW = NC * NS
    NJ = B0 // NW
    CH = B1
    assert NJ * NW == B0 and D % L == 0 and NJ >= 4 and NJ % 2 == 0

    xr = x.reshape(NW, NJ, CH).astype(jnp.int32)
    mesh = plsc.VectorSubcoreMesh(core_axis_name="c", subcore_axis_name="s")

    @functools.partial(
        pl.kernel,
        mesh=mesh,
        out_type=jax.ShapeDtypeStruct((B0, B1, D), jnp.float32),
        scratch_types=[
            pltpu.VMEM((NJ, CH), jnp.int32),
            pltpu.VMEM((2, CH, D), jnp.float32),
            pltpu.VMEM((2, CH, D), jnp.float32),
            pltpu.SemaphoreType.DMA,
            pltpu.SemaphoreType.DMA,
            pltpu.SemaphoreType.DMA,
            pltpu.SemaphoreType.DMA,
        ],
    )
    def emb_kernel(x_hbm, table_hbm, out_hbm, idx_v, rows_in, rows_out,
                   sem_g0, sem_g1, sem_s0, sem_s1):
        c = lax.axis_index("c")
        s = lax.axis_index("s")
        wid = s * NC + c
        row0 = wid * NJ
        pltpu.sync_copy(x_hbm.at[wid], idx_v)
        sem_g = (sem_g0, sem_g1)
        sem_s = (sem_s0, sem_s1)

        def g_copy(jj, b):
            return pltpu.make_async_copy(
                table_hbm.at[idx_v.at[jj]], rows_in.at[b], sem_g[b])

        def s_copy(jj, b):
            return pltpu.make_async_copy(
                rows_out.at[b], out_hbm.at[row0 + jj], sem_s[b])

        def scale(b):
            @plsc.parallel_loop(0, CH, unroll=4)
            def _(r):
                for t in range(D // 16):
                    sl = pl.ds(t * 16, 16)
                    rows_out[b, r, sl] = rows_in[b, r, sl] * SCALE

        g_copy(0, 0).start()
        g_copy(0, 0).wait()
        g_copy(1, 1).start()
        scale(0)
        s_copy(0, 0).start()
        g_copy(1, 1).wait()
        g_copy(2, 0).start()
        scale(1)
        s_copy(1, 1).start()

        def pair(p, carry):
            jj0 = 2 * p + 2
            for b in range(2):
                jj = jj0 + b
                g_copy(jj, b).wait()
                s_copy(jj - 2, b).wait()
                g_copy(jj + 1, 1 - b).start()
                scale(b)
                s_copy(jj, b).start()
            return carry
        lax.fori_loop(0, (NJ - 4) // 2, pair, 0)

        g_copy(NJ - 2, 0).wait()
        s_copy(NJ - 4, 0).wait()
        g_copy(NJ - 1, 1).start()
        scale(0)
        s_copy(NJ - 2, 0).start()
        g_copy(NJ - 1, 1).wait()
        s_copy(NJ - 3, 1).wait()
        scale(1)
        s_copy(NJ - 1, 1).start()
        s_copy(NJ - 2, 0).wait()
        s_copy(NJ - 1, 1).wait()

    return emb_kernel(xr, table)


# speedup vs baseline: 1.8173x; 1.3925x over previous
import functools
import math

import jax
import jax.numpy as jnp
from jax import lax
from jax.experimental import pallas as pl
from jax.experimental.pallas import tpu as pltpu
from jax.experimental.pallas import tpu_sc as plsc

D_MODEL = 128
SCALE = math.sqrt(float(D_MODEL))


def kernel(x, table):
    B0, B1 = x.shape
    V, D = table.shape
    info = plsc.get_sparse_core_info()
    NC, NS, L = info.num_cores, info.num_subcores, info.num_lanes
    NW = NC * NS
    ROWS_W = B0 // NW
    CH0 = 4
    GR = 2 * B1
    NG = (CH0 * B1) // GR
    NJ = ROWS_W // CH0
    assert ROWS_W * NW == B0 and D % L == 0 and NJ >= 4 and NJ % 2 == 0
    assert NG * GR == CH0 * B1 and GR <= 128

    xr = x.reshape(NW, NJ, NG, GR).astype(jnp.int32)
    mesh = plsc.VectorSubcoreMesh(core_axis_name="c", subcore_axis_name="s")

    @functools.partial(
        pl.kernel,
        mesh=mesh,
        out_type=jax.ShapeDtypeStruct((B0, B1, D), jnp.float32),
        scratch_types=[
            pltpu.VMEM((NJ, NG, GR), jnp.int32),
            pltpu.VMEM((2, NG * GR, D), jnp.float32),
            pltpu.VMEM((2, CH0, B1, D), jnp.float32),
            pltpu.SemaphoreType.DMA,
            pltpu.SemaphoreType.DMA,
            pltpu.SemaphoreType.DMA,
            pltpu.SemaphoreType.DMA,
        ],
    )
    def emb_kernel(x_hbm, table_hbm, out_hbm, idx_v, rows_in, rows_out,
                   sem_g0, sem_g1, sem_s0, sem_s1):
        c = lax.axis_index("c")
        s = lax.axis_index("s")
        wid = s * NC + c
        row0 = wid * ROWS_W
        pltpu.sync_copy(x_hbm.at[wid], idx_v)
        sem_g = (sem_g0, sem_g1)
        sem_s = (sem_s0, sem_s1)

        def g_copies(jj, b):
            return [
                pltpu.make_async_copy(
                    table_hbm.at[idx_v.at[jj, g]],
                    rows_in.at[b, pl.ds(g * GR, GR)],
                    sem_g[b])
                for g in range(NG)
            ]

        def g_start(jj, b):
            for cp in g_copies(jj, b):
                cp.start()

        def g_wait(jj, b):
            for cp in g_copies(jj, b):
                cp.wait()

        def s_copy(jj, b):
            return pltpu.make_async_copy(
                rows_out.at[b], out_hbm.at[pl.ds(row0 + jj * CH0, CH0)],
                sem_s[b])

        def scale(b):
            @plsc.parallel_loop(0, B1, unroll=2)
            def _(r):
                for rr in range(CH0):
                    for t in range(D // 16):
                        sl = pl.ds(t * 16, 16)
                        rows_out[b, rr, r, sl] = rows_in[b, rr * B1 + r, sl] * SCALE

        g_start(0, 0)
        g_wait(0, 0)
        g_start(1, 1)
        scale(0)
        s_copy(0, 0).start()
        g_wait(1, 1)
        g_start(2, 0)
        scale(1)
        s_copy(1, 1).start()

        def pair(p, carry):
            jj0 = 2 * p + 2
            for b in range(2):
                jj = jj0 + b
                g_wait(jj, b)
                s_copy(jj - 2, b).wait()
                g_start(jj + 1, 1 - b)
                scale(b)
                s_copy(jj, b).start()
            return carry
        lax.fori_loop(0, (NJ - 4) // 2, pair, 0)

        g_wait(NJ - 2, 0)
        s_copy(NJ - 4, 0).wait()
        g_start(NJ - 1, 1)
        scale(0)
        s_copy(NJ - 2, 0).start()
        g_wait(NJ - 1, 1)
        s_copy(NJ - 3, 1).wait()
        scale(1)
        s_copy(NJ - 1, 1).start()
        s_copy(NJ - 2, 0).wait()
        s_copy(NJ - 1, 1).wait()

    return emb_kernel(xr, table)
